# baseline (speedup 1.0000x reference)
import functools

import jax
import jax.numpy as jnp
from jax import lax
from jax.experimental import pallas as pl
from jax.experimental.pallas import tpu as pltpu
from jax.experimental.pallas import tpu_sc as plsc

N = 10000
NP = 10240
E = 320000
EP = 321536
D = 32
NC = 2
NS = 16
BB = 128
EPT = EP // NS
NCHUNK = EPT // BB
EPT32 = EP // (NC * NS)
B1 = 64
NCHUNK1 = EPT32 // B1
NG3 = EPT32 // 16
RPT = N // NS
ROWCHUNKS = ((0, 128), (128, 128), (256, 128), (384, 128), (512, 113))

_mesh = functools.partial(
    plsc.VectorSubcoreMesh, core_axis_name="c", subcore_axis_name="s")

_sc_params = pltpu.CompilerParams(
    needs_layout_passes=False, use_tc_tiling_on_sc=False)


def _zero_vmem_2d(ref, nrows, ncols):
    def body(e, _):
        for j in range(ncols // 16):
            ref[e, pl.ds(j * 16, 16)] = jnp.zeros((16,), jnp.float32)
        return 0
    lax.fori_loop(0, nrows, body, 0)


def _zero_vmem_1d(ref, n):
    def body(g, _):
        ref[pl.ds(g * 16, 16)] = jnp.zeros((16,), jnp.float32)
        return 0
    lax.fori_loop(0, n // 16, body, 0)


def _deg_body(row2_hbm, w_hbm, out_hbm, rowstage, wstage, degloc, deg_sh):
    c = lax.axis_index("c")
    s = lax.axis_index("s")
    wid = s * NC + c

    _zero_vmem_1d(degloc, NP)
    pltpu.sync_copy(degloc.at[pl.ds(0, NP // NS)],
                    deg_sh.at[pl.ds(s * (NP // NS), NP // NS)])
    plsc.subcore_barrier()

    pltpu.sync_copy(row2_hbm.at[wid], rowstage)
    pltpu.sync_copy(w_hbm.at[pl.ds(wid * EPT32, EPT32)], wstage)

    def chunk(k, _):
        pltpu.sync_copy(wstage.at[pl.ds(k * B1, B1)],
                        deg_sh.at[rowstage.at[k]], add=True)
        return 0
    lax.fori_loop(0, NCHUNK1, chunk, 0)

    plsc.subcore_barrier()

    @pl.when(s == 0)
    def _():
        pltpu.sync_copy(deg_sh, degloc)
        pltpu.sync_copy(degloc, out_hbm.at[c])


_deg_call = pl.kernel(
    _deg_body,
    out_type=jax.ShapeDtypeStruct((NC, NP), jnp.float32),
    mesh=_mesh(),
    compiler_params=_sc_params,
    scratch_types=[
        pltpu.VMEM((NCHUNK1, B1), jnp.int32),
        pltpu.VMEM((EPT32,), jnp.float32),
        pltpu.VMEM((NP,), jnp.float32),
        pltpu.VMEM_SHARED((NP,), jnp.float32),
    ],
)


def _dis_body(degp_ref, dis_ref):
    deg = degp_ref[0] + degp_ref[1]
    safe = jnp.where(deg > 0, deg, 1.0)
    dis_ref[...] = jnp.where(deg > 0, lax.rsqrt(safe), 0.0)


def _dis_call(degp):
    return pl.pallas_call(
        _dis_body,
        out_shape=jax.ShapeDtypeStruct((NP // 128, 128), jnp.float32),
    )(degp.reshape(NC, NP // 128, 128))


def _norm_body(row_hbm, col_hbm, w_hbm, dis_hbm, norm_hbm,
               rstage, cstage, wstage, disloc, normloc):
    c = lax.axis_index("c")
    s = lax.axis_index("s")
    wid = s * NC + c
    base = wid * EPT32

    pltpu.sync_copy(dis_hbm, disloc)
    pltpu.sync_copy(row_hbm.at[pl.ds(base, EPT32)], rstage)
    pltpu.sync_copy(col_hbm.at[pl.ds(base, EPT32)], cstage)
    pltpu.sync_copy(w_hbm.at[pl.ds(base, EPT32)], wstage)

    def grp(g, _):
        rv = rstage[pl.ds(g * 16, 16)]
        cv = cstage[pl.ds(g * 16, 16)]
        wv = wstage[pl.ds(g * 16, 16)]
        dr = plsc.load_gather(disloc, [rv])
        dc = plsc.load_gather(disloc, [cv])
        normloc[pl.ds(g * 16, 16)] = -(dr * wv * dc)
        return 0
    lax.fori_loop(0, NG3, grp, 0)

    pltpu.sync_copy(normloc, norm_hbm.at[pl.ds(base, EPT32)])


_norm_call = pl.kernel(
    _norm_body,
    out_type=jax.ShapeDtypeStruct((EP,), jnp.float32),
    mesh=_mesh(),
    compiler_params=_sc_params,
    scratch_types=[
        pltpu.VMEM((EPT32,), jnp.int32),
        pltpu.VMEM((EPT32,), jnp.int32),
        pltpu.VMEM((EPT32,), jnp.float32),
        pltpu.VMEM((NP,), jnp.float32),
        pltpu.VMEM((EPT32,), jnp.float32),
    ],
)


def _spmm_body(z_hbm, col_hbm, row2_hbm, norm_hbm, out_hbm,
               colstage, rowstage, normstage, rows0, rows1,
               gsem0, gsem1, ssem0, ssem1, acc_sh, z_sh):
    c = lax.axis_index("c")
    s = lax.axis_index("s")
    r0 = s * RPT

    base = s * EPT
    pltpu.sync_copy(col_hbm.at[pl.ds(base, EPT)], colstage)
    pltpu.sync_copy(row2_hbm.at[s], rowstage)
    pltpu.sync_copy(norm_hbm.at[pl.ds(base, EPT)], normstage)

    def z_src(k):
        return z_sh.at[colstage.at[pl.ds(k * BB, BB)]]

    def issue_gather(k, buf, sem):
        pltpu.async_copy(z_src(k), buf, sem)

    def wait_gather(k, buf, sem):
        pltpu.make_async_copy(z_src(k), buf, sem).wait()

    def issue_scatter(k, buf, sem):
        pltpu.async_copy(buf, acc_sh.at[rowstage.at[k]], sem, add=True)

    def wait_scatter(k, buf, sem):
        pltpu.make_async_copy(buf, acc_sh.at[rowstage.at[k]], sem).wait()

    def scale(buf, k):
        def grp(g, _):
            nv = normstage[pl.ds(k * BB + g * 16, 16)]
            for l in range(16):
                e = g * 16 + l
                sv = nv[l]
                for j in range(D // 16):
                    buf[e, pl.ds(j * 16, 16)] = buf[e, pl.ds(j * 16, 16)] * sv
            return 0
        lax.fori_loop(0, BB // 16, grp, 0)

    for p in (0, 1):
        _zero_vmem_2d(rows0, BB, D)
        for (off, sz) in ROWCHUNKS:
            pltpu.sync_copy(rows0.at[pl.ds(0, sz)],
                            acc_sh.at[pl.ds(r0 + off, sz)])
        for (off, sz) in ROWCHUNKS:
            pltpu.sync_copy(z_hbm.at[p].at[pl.ds(c * N + r0 + off, sz)],
                            rows1.at[pl.ds(0, sz)])
            pltpu.sync_copy(rows1.at[pl.ds(0, sz)],
                            z_sh.at[pl.ds(r0 + off, sz)])
        plsc.subcore_barrier()

        issue_gather(0, rows0, gsem0)

        def pair(q, _):
            k0 = 2 * q
            wait_gather(k0, rows0, gsem0)

            @pl.when(q >= 1)
            def _():
                wait_scatter(k0 - 1, rows1, ssem1)
            issue_gather(k0 + 1, rows1, gsem1)
            scale(rows0, k0)
            issue_scatter(k0, rows0, ssem0)

            wait_gather(k0 + 1, rows1, gsem1)
            wait_scatter(k0, rows0, ssem0)
            issue_gather(k0 + 2, rows0, gsem0)
            scale(rows1, k0 + 1)
            issue_scatter(k0 + 1, rows1, ssem1)
            return 0
        lax.fori_loop(0, (NCHUNK - 1) // 2, pair, 0)

        klast = NCHUNK - 1
        wait_gather(klast, rows0, gsem0)
        wait_scatter(klast - 1, rows1, ssem1)
        scale(rows0, klast)
        issue_scatter(klast, rows0, ssem0)
        wait_scatter(klast, rows0, ssem0)

        plsc.subcore_barrier()

        for (off, sz) in ROWCHUNKS:
            pltpu.sync_copy(acc_sh.at[pl.ds(r0 + off, sz)],
                            rows0.at[pl.ds(0, sz)])
            pltpu.sync_copy(rows0.at[pl.ds(0, sz)],
                            out_hbm.at[p].at[pl.ds(c * N + r0 + off, sz)])
        plsc.subcore_barrier()


_spmm_call = pl.kernel(
    _spmm_body,
    out_type=jax.ShapeDtypeStruct((2, NC * N, D), jnp.float32),
    mesh=_mesh(),
    compiler_params=_sc_params,
    scratch_types=[
        pltpu.VMEM((EPT,), jnp.int32),
        pltpu.VMEM((NCHUNK, BB), jnp.int32),
        pltpu.VMEM((EPT,), jnp.float32),
        pltpu.VMEM((BB, D), jnp.float32),
        pltpu.VMEM((BB, D), jnp.float32),
        pltpu.SemaphoreType.DMA,
        pltpu.SemaphoreType.DMA,
        pltpu.SemaphoreType.DMA,
        pltpu.SemaphoreType.DMA,
        pltpu.VMEM_SHARED((N, D), jnp.float32),
        pltpu.VMEM_SHARED((N, D), jnp.float32),
    ],
)


def _cat128(lo, hi):
    return jnp.concatenate([lo[0], lo[1], hi[0], hi[1]], axis=1)


def _dense_body(act, xlo, xhi, t1lo, t1hi, ulo, uhi, a_ref, b_ref, c_ref,
                bias_ref, out_lo, out_hi):
    x = _cat128(xlo[...], xhi[...])
    t1 = _cat128(t1lo[...], t1hi[...])
    u = _cat128(ulo[...], uhi[...])
    acc = jnp.dot(x, a_ref[...], preferred_element_type=jnp.float32)
    acc = acc + jnp.dot(t1, b_ref[...], preferred_element_type=jnp.float32)
    acc = acc + jnp.dot(u, c_ref[...], preferred_element_type=jnp.float32)
    acc = acc + bias_ref[...]
    if act == "relu":
        acc = jnp.maximum(acc, 0.0)
    elif act == "logsoftmax":
        m = jnp.max(acc, axis=1, keepdims=True)
        acc = acc - m
        acc = acc - jnp.log(jnp.sum(jnp.exp(acc), axis=1, keepdims=True))
    out_lo[0] = acc[:, 0:32]
    out_lo[1] = acc[:, 32:64]
    out_hi[0] = acc[:, 64:96]
    out_hi[1] = acc[:, 96:128]


def _dense_call(x, t1, u, a, b, c, bias, act):
    blk = 400
    grid = N // blk
    lo_spec = pl.BlockSpec((2, blk, D), lambda i: (0, i, 0))
    hi_spec = pl.BlockSpec((2, blk, D), lambda i: (0, N // blk + i, 0))
    w_spec = pl.BlockSpec((128, 128), lambda i: (0, 0))
    out_lo, out_hi = pl.pallas_call(
        functools.partial(_dense_body, act),
        grid=(grid,),
        in_specs=[lo_spec, hi_spec, lo_spec, hi_spec, lo_spec, hi_spec,
                  w_spec, w_spec, w_spec,
                  pl.BlockSpec((1, 128), lambda i: (0, 0))],
        out_specs=[pl.BlockSpec((2, blk, D), lambda i: (0, i, 0)),
                   pl.BlockSpec((2, blk, D), lambda i: (0, i, 0))],
        out_shape=[jax.ShapeDtypeStruct((2, N, D), jnp.float32),
                   jax.ShapeDtypeStruct((2, N, D), jnp.float32)],
    )(x, x, t1, t1, u, u, a, b, c, bias.reshape(1, 128))
    return jnp.concatenate([out_lo, out_hi], axis=1)


def kernel(data, edge_index, edgenet_input, W1, b1, W2, b2, W3, b3):
    w = edgenet_input[:, 0]
    row = edge_index[0]
    col = edge_index[1]

    pad = EP - E
    roww = jnp.concatenate([row, jnp.zeros((pad,), jnp.int32)])
    colw = jnp.concatenate([col, jnp.zeros((pad,), jnp.int32)])
    ww = jnp.concatenate([w, jnp.zeros((pad,), jnp.float32)])

    row_k1 = roww.reshape(NC * NS, NCHUNK1, B1)
    row_spmm = roww.reshape(NS, NCHUNK, BB)

    degp = _deg_call(row_k1, ww)
    dis = _dis_call(degp).reshape(-1)
    norm = _norm_call(roww, colw, ww, dis)

    h = jnp.stack([
        jnp.concatenate([data[:, 0:32], data[:, 64:96]]),
        jnp.concatenate([data[:, 32:64], data[:, 96:128]]),
    ])

    for (W, b, act) in ((W1, b1, "relu"), (W2, b2, "relu"),
                        (W3, b3, "logsoftmax")):
        a_w = W[0] - W[2]
        b_w = W[1]
        c_w = 2.0 * W[2]
        t1 = _spmm_call(h, colw, row_spmm, norm)
        u = _spmm_call(t1, colw, row_spmm, norm)
        h = _dense_call(h, t1, u, a_w, b_w, c_w, b, act)

    return jnp.concatenate([h[0, :N], h[1, :N], h[0, N:], h[1, N:]], axis=1)
